# baseline (speedup 1.0000x reference)
import functools
import math

import jax
import jax.numpy as jnp
from jax.experimental import pallas as pl
from jax.experimental.pallas import tpu as pltpu

_EMBED_DIM = 64
_LOG10000 = math.log(10000.0)


def _body(var_ref, out_ref, pe_ref, *, T, D):
    E = _EMBED_DIM
    pos = jax.lax.broadcasted_iota(jnp.int32, (T, 128), 0).astype(jnp.float32)
    lane = jax.lax.broadcasted_iota(jnp.int32, (T, 128), 1)
    k = ((lane & (E - 1)) >> 1).astype(jnp.float32)
    freq = jnp.exp(k * (-2.0 * _LOG10000 / E))
    angle = pos * freq
    pe_ref[...] = jnp.where(lane & 1 == 0, jnp.sin(angle), jnp.cos(angle))
    out_ref[0] = var_ref[...][None, :, :] + pe_ref[...][:, None, :]


def kernel(x, var_table):
    B, T, D = x.shape
    E = _EMBED_DIM
    var_packed = var_table.reshape(D // 2, 2 * E)
    s = pl.pallas_call(
        functools.partial(_body, T=T, D=D),
        in_specs=[pl.BlockSpec((D // 2, 2 * E), lambda: (0, 0))],
        out_specs=pl.BlockSpec((1, T, D // 2, 2 * E), lambda: (0, 0, 0, 0)),
        out_shape=jax.ShapeDtypeStruct((1, T, D // 2, 2 * E), jnp.float32),
        scratch_shapes=[pltpu.VMEM((T, 2 * E), jnp.float32)],
    )(var_packed)
    return jnp.broadcast_to(s.reshape(1, T, D, E), (B, T, D, E))


# speedup vs baseline: 1.1124x; 1.1124x over previous
import functools
import math

import jax
import jax.numpy as jnp
from jax.experimental import pallas as pl
from jax.experimental.pallas import tpu as pltpu

_EMBED_DIM = 64
_LOG10000 = math.log(10000.0)


def _body(var_ref, out_ref, *, chunk, D):
    E = _EMBED_DIM
    t0 = pl.program_id(0) * chunk
    pos = (t0 + jax.lax.broadcasted_iota(jnp.int32, (chunk, E), 0)).astype(
        jnp.float32)
    e_idx = jax.lax.broadcasted_iota(jnp.int32, (chunk, E), 1)
    k = (e_idx >> 1).astype(jnp.float32)
    freq = jnp.exp(k * (-2.0 * _LOG10000 / E))
    angle = pos * freq
    pe = jnp.where(e_idx & 1 == 0, jnp.sin(angle), jnp.cos(angle))
    out_ref[0] = var_ref[...][None, :, :] + pe[:, None, :]


def kernel(x, var_table):
    B, T, D = x.shape
    E = _EMBED_DIM
    chunk = 40
    s = pl.pallas_call(
        functools.partial(_body, chunk=chunk, D=D),
        grid=(T // chunk,),
        in_specs=[pl.BlockSpec((D, E), lambda i: (0, 0))],
        out_specs=pl.BlockSpec((1, chunk, D, E), lambda i: (0, i, 0, 0)),
        out_shape=jax.ShapeDtypeStruct((1, T, D, E), jnp.float32),
    )(var_table)
    return jnp.broadcast_to(s, (B, T, D, E))
